# baseline (speedup 1.0000x reference)
import functools

import jax
import jax.numpy as jnp
from jax import lax
from jax.experimental import pallas as pl
from jax.experimental.pallas import tpu as pltpu
from jax.experimental.pallas import tpu_sc as plsc

NC = 2
NS = 16
L = 16
D = 128
CB = 16
CS = 8
C = CB * CS
NBUF = 5
RU = 4


@functools.partial(jax.jit, static_argnums=(0, 1))
def _sc_joint_embedding(batch, seq, idx_b, tok, seg, pos):
    nw = NC * NS
    n_chunks = (batch // nw) * seq // C
    nsb = seq // CS
    half = seq // 2
    mesh = plsc.VectorSubcoreMesh(core_axis_name="c", subcore_axis_name="s")

    @functools.partial(
        pl.kernel,
        out_type=jax.ShapeDtypeStruct((batch, seq, D), jnp.float32),
        mesh=mesh,
        scratch_types=[
            pltpu.VMEM((seq, D), jnp.float32),
            pltpu.VMEM((2, D), jnp.float32),
            pltpu.VMEM((NBUF, C), jnp.int32),
            pltpu.VMEM((NBUF, CB, CS, D), jnp.float32),
            [pltpu.SemaphoreType.DMA] * NBUF,
            [pltpu.SemaphoreType.DMA] * NBUF,
            [pltpu.SemaphoreType.DMA] * NBUF,
        ],
    )
    def k(idx_hbm, tok_hbm, seg_hbm, pos_hbm, out_hbm,
          posb, segv, idxv, rows, isem, gsem, ssem):
        wid = lax.axis_index("s") * NC + lax.axis_index("c")
        wb = wid * (batch // nw)

        pltpu.sync_copy(pos_hbm.at[pl.ds(0, seq)], posb)
        pltpu.sync_copy(seg_hbm.at[pl.ds(0, 2)], segv)

        def idx_copy(t, b):
            return pltpu.make_async_copy(
                idx_hbm.at[pl.ds((wid * n_chunks + t) * C, C)],
                idxv.at[b], isem[b])

        def gat_copy(t, b):
            return pltpu.make_async_copy(
                tok_hbm.at[idxv.at[b]], rows.at[b].reshape(C, D), gsem[b])

        def scat_copy(t, b):
            b0 = wb + (t // nsb) * CB
            s0 = lax.rem(t, nsb) * CS
            return pltpu.make_async_copy(
                rows.at[b],
                out_hbm.at[pl.ds(b0, CB), pl.ds(s0, CS)], ssem[b])

        for t in range(NBUF - 1):
            idx_copy(t, t).start()
        for t in range(NBUF - 2):
            idx_copy(t, t).wait()
            gat_copy(t, t).start()

        def group(g, carry):
            for b in range(NBUF):
                j = g * NBUF + b
                ti = j + NBUF - 1
                bi = (b + NBUF - 1) % NBUF

                @pl.when(ti < n_chunks)
                def _():
                    idx_copy(ti, bi).start()

                tg = j + NBUF - 2
                bg = (b + NBUF - 2) % NBUF

                @pl.when(tg < n_chunks)
                def _():
                    @pl.when(j >= 2)
                    def _():
                        scat_copy(j - 2, bg).wait()
                    idx_copy(tg, bg).wait()
                    gat_copy(tg, bg).start()

                gat_copy(j, b).wait()

                s0 = lax.rem(j, nsb) * CS
                for sl in range(CS):
                    s = s0 + sl
                    srow = jnp.where(s > half, 1, 0)
                    bias_c = [posb[s, pl.ds(c * L, L)]
                              + segv[srow, pl.ds(c * L, L)]
                              for c in range(D // L)]

                    def add_bias(r, inner):
                        base = r * RU
                        for k in range(RU):
                            for c in range(D // L):
                                plsc.addupdate(
                                    rows.at[b, base + k, sl, pl.ds(c * L, L)],
                                    bias_c[c])
                        return inner
                    lax.fori_loop(0, CB // RU, add_bias, 0)

                scat_copy(j, b).start()
            return carry

        lax.fori_loop(0, n_chunks // NBUF, group, 0)

        for b in range(NBUF):
            scat_copy(n_chunks - NBUF + b, b).wait()

    return k(idx_b, tok, seg, pos)


def kernel(input_tensor, token_emb, segment_emb, positional_emb):
    b, s = input_tensor.shape
    idx_b = (input_tensor
             .reshape(b // CB, CB, s // CS, CS)
             .transpose(0, 2, 1, 3)
             .reshape(b * s))
    return _sc_joint_embedding(b, s, idx_b, token_emb,
                               segment_emb, positional_emb)


# speedup vs baseline: 1.3641x; 1.3641x over previous
import functools

import jax
import jax.numpy as jnp
from jax import lax
from jax.experimental import pallas as pl
from jax.experimental.pallas import tpu as pltpu
from jax.experimental.pallas import tpu_sc as plsc

NC = 2
NS = 16
L = 16
D = 128
C = 128
NBUF = 4
RU = 8


@functools.partial(jax.jit, static_argnums=(0, 1))
def _sc_joint_embedding(batch, seq, idx_t, tok, seg, pos):
    nw = NC * NS
    n_chunks = seq
    half = seq // 2
    mesh = plsc.VectorSubcoreMesh(core_axis_name="c", subcore_axis_name="s")

    @functools.partial(
        pl.kernel,
        out_type=jax.ShapeDtypeStruct((batch, seq, D), jnp.float32),
        mesh=mesh,
        scratch_types=[
            pltpu.VMEM((seq, D), jnp.float32),
            pltpu.VMEM((2, D), jnp.float32),
            pltpu.VMEM((seq, C), jnp.int32),
            pltpu.VMEM((NBUF, C, D), jnp.float32),
            [pltpu.SemaphoreType.DMA] * NBUF,
            [pltpu.SemaphoreType.DMA] * NBUF,
        ],
    )
    def k(idx_hbm, tok_hbm, seg_hbm, pos_hbm, out_hbm,
          posb, segv, idxa, rows, gsem, ssem):
        wid = lax.axis_index("s") * NC + lax.axis_index("c")
        wb = wid * C

        pltpu.sync_copy(idx_hbm.at[pl.ds(0, seq), pl.ds(wb, C)], idxa)
        pltpu.sync_copy(pos_hbm.at[pl.ds(0, seq)], posb)
        pltpu.sync_copy(seg_hbm.at[pl.ds(0, 2)], segv)

        def gat_copy(t, b):
            return pltpu.make_async_copy(
                tok_hbm.at[idxa.at[t]], rows.at[b], gsem[b])

        def scat_copy(t, b):
            return pltpu.make_async_copy(
                rows.at[b], out_hbm.at[pl.ds(wb, C), t], ssem[b])

        for t in range(NBUF - 2):
            gat_copy(t, t).start()

        def group(g, carry):
            for b in range(NBUF):
                j = g * NBUF + b
                tg = j + NBUF - 2
                bg = (b + NBUF - 2) % NBUF

                @pl.when(tg < n_chunks)
                def _():
                    @pl.when(j >= 2)
                    def _():
                        scat_copy(j - 2, bg).wait()
                    gat_copy(tg, bg).start()

                gat_copy(j, b).wait()

                srow = jnp.where(j > half, 1, 0)
                bias_c = [posb[j, pl.ds(c * L, L)] + segv[srow, pl.ds(c * L, L)]
                          for c in range(D // L)]

                def add_bias(r, inner):
                    base = r * RU
                    for k in range(RU):
                        for c in range(D // L):
                            plsc.addupdate(
                                rows.at[b, base + k, pl.ds(c * L, L)],
                                bias_c[c])
                    return inner
                lax.fori_loop(0, C // RU, add_bias, 0)

                scat_copy(j, b).start()
            return carry

        lax.fori_loop(0, n_chunks // NBUF, group, 0)

        for b in range(NBUF):
            scat_copy(n_chunks - NBUF + b, b).wait()

    return k(idx_t, tok, seg, pos)


def kernel(input_tensor, token_emb, segment_emb, positional_emb):
    b, s = input_tensor.shape
    idx_t = input_tensor.T
    return _sc_joint_embedding(b, s, idx_t, token_emb,
                               segment_emb, positional_emb)
